# baseline (speedup 1.0000x reference)
import functools

import jax
import jax.numpy as jnp
from jax import lax
from jax.experimental import pallas as pl
from jax.experimental.pallas import tpu as pltpu
from jax.experimental.pallas import tpu_sc as plsc

_E = 8
_K = 2
_H = 1024
_F = 704
_FS = 1408
_T = 2048
_NP = _T * _K
_BM = 64
_NPAD = _NP + _E * _BM
_NT = _NPAD // _BM
_NTP = 80

_NC = 2
_NS = 16
_NW = _NC * _NS
_RW = _NPAD // _NW
_PFW = _NPAD // _NS
_SCW = _NP // _NS


def _gate_body(x_ref, gw_ref, pos_ref, wts_ref, teid_ref, s_ref):
    x = x_ref[...]
    gw = gw_ref[...]
    logits = jax.lax.dot_general(x, gw, (((1,), (1,)), ((), ())),
                                 preferred_element_type=jnp.float32)
    m = jnp.max(logits, axis=-1, keepdims=True)
    ex = jnp.exp(logits - m)
    scores = ex / jnp.sum(ex, axis=-1, keepdims=True)
    cols = jax.lax.broadcasted_iota(jnp.int32, scores.shape, 1)
    m1 = jnp.max(scores, axis=-1, keepdims=True)
    i1 = jnp.min(jnp.where(scores == m1, cols, _E), axis=-1, keepdims=True)
    masked = jnp.where(cols == i1, -jnp.inf, scores)
    m2 = jnp.max(masked, axis=-1, keepdims=True)
    i2 = jnp.min(jnp.where(masked == m2, cols, _E), axis=-1, keepdims=True)
    denom = m1 + m2 + 1e-20
    wts_ref[...] = jnp.concatenate([m1 / denom, m2 / denom], axis=1)

    oh0 = jnp.where(cols == i1, 1.0, 0.0)
    oh1 = jnp.where(cols == i2, 1.0, 0.0)
    both = oh0 + oh1

    cb = 128
    rr = jax.lax.broadcasted_iota(jnp.int32, (cb, cb), 0)
    cc = jax.lax.broadcasted_iota(jnp.int32, (cb, cb), 1)
    tri = jnp.where(rr > cc, 1.0, 0.0)
    off = jnp.zeros((1, _E), jnp.float32)
    for b in range(_T // cb):
        blk = both[b * cb:(b + 1) * cb, :]
        s_ref[pl.ds(b * cb, cb), :] = jax.lax.dot_general(
            tri, blk, (((1,), (0,)), ((), ())),
            preferred_element_type=jnp.float32) + off
        off = off + jnp.sum(blk, axis=0, keepdims=True)
    s = s_ref[...]

    counts = off
    padded = jnp.floor((counts + (_BM - 1)) / _BM) * _BM
    er = jax.lax.broadcasted_iota(jnp.int32, (_E, _E), 0)
    ec = jax.lax.broadcasted_iota(jnp.int32, (_E, _E), 1)
    triu = jnp.where(er <= ec, 1.0, 0.0)
    ends = jax.lax.dot_general(padded, triu, (((1,), (0,)), ((), ())),
                               preferred_element_type=jnp.float32)
    starts = ends - padded

    pos0 = jnp.sum(oh0 * (starts + s), axis=1, keepdims=True)
    pos1 = jnp.sum(oh1 * (starts + s + oh0), axis=1, keepdims=True)
    pos_ref[...] = jnp.concatenate([pos0, pos1], axis=1).astype(jnp.int32)

    tstart = (jax.lax.broadcasted_iota(jnp.int32, (_NTP, _E), 0)
              * _BM).astype(jnp.float32)
    acc = jnp.sum(jnp.where(tstart >= ends, 1.0, 0.0), axis=1)
    teid_ref[...] = jnp.minimum(acc, _E - 1).astype(jnp.int32).reshape(1, _NTP)


def _gate_route(x, gate_w):
    return pl.pallas_call(
        _gate_body,
        out_shape=(jax.ShapeDtypeStruct((_T, _K), jnp.int32),
                   jax.ShapeDtypeStruct((_T, _K), jnp.float32),
                   jax.ShapeDtypeStruct((1, _NTP), jnp.int32)),
        scratch_shapes=[pltpu.VMEM((_T, _E), jnp.float32)],
    )(x, gate_w)


_TW = _T // _NW


def _dispatch_body(p0_hbm, p1_hbm, x_hbm, xs_out, idx0_v, idx1_v, rows_v,
                   sem0, sem1):
    wid = lax.axis_index("c") * _NS + lax.axis_index("s")
    tb = wid * _TW
    pltpu.sync_copy(p0_hbm.at[pl.ds(tb, _TW)], idx0_v)
    pltpu.sync_copy(p1_hbm.at[pl.ds(tb, _TW)], idx1_v)
    pltpu.sync_copy(x_hbm.at[pl.ds(tb, _TW)], rows_v)
    d0 = pltpu.async_copy(rows_v, xs_out.at[idx0_v], sem0)
    d1 = pltpu.async_copy(rows_v, xs_out.at[idx1_v], sem1)
    d0.wait()
    d1.wait()


def _dispatch(pos0, pos1, x):
    mesh = plsc.VectorSubcoreMesh(core_axis_name="c", subcore_axis_name="s",
                                  num_cores=_NC, num_subcores=_NS)
    return pl.kernel(
        _dispatch_body,
        out_type=jax.ShapeDtypeStruct((_NPAD, _H), jnp.float32),
        mesh=mesh,
        scratch_types=[
            pltpu.VMEM((_TW,), jnp.int32),
            pltpu.VMEM((_TW,), jnp.int32),
            pltpu.VMEM((_TW, _H), jnp.float32),
            pltpu.SemaphoreType.DMA,
            pltpu.SemaphoreType.DMA,
        ],
        name="moe_dispatch_sc",
    )(pos0, pos1, x)


def _moe_body(tile_eid_ref, xs_ref, wg_ref, wu_ref, wd_ref, out_ref):
    x = xs_ref[...]
    g = jax.lax.dot_general(x, wg_ref[0], (((1,), (1,)), ((), ())),
                            preferred_element_type=jnp.float32)
    u = jax.lax.dot_general(x, wu_ref[0], (((1,), (1,)), ((), ())),
                            preferred_element_type=jnp.float32)
    h = g * jax.nn.sigmoid(g) * u
    out_ref[...] = jax.lax.dot_general(h, wd_ref[0], (((1,), (1,)), ((), ())),
                                       preferred_element_type=jnp.float32)


def _moe_mlp(xs, w_gate, w_up, w_down, tile_eid):
    grid_spec = pltpu.PrefetchScalarGridSpec(
        num_scalar_prefetch=1,
        grid=(_NT,),
        in_specs=[
            pl.BlockSpec((_BM, _H), lambda i, eid: (i, 0)),
            pl.BlockSpec((1, _F, _H), lambda i, eid: (eid[i], 0, 0)),
            pl.BlockSpec((1, _F, _H), lambda i, eid: (eid[i], 0, 0)),
            pl.BlockSpec((1, _H, _F), lambda i, eid: (eid[i], 0, 0)),
        ],
        out_specs=pl.BlockSpec((_BM, _H), lambda i, eid: (i, 0)),
    )
    return pl.pallas_call(
        _moe_body,
        grid_spec=grid_spec,
        out_shape=jax.ShapeDtypeStruct((_NPAD, _H), jnp.float32),
    )(tile_eid, xs, w_gate, w_up, w_down)


def _shared_body(x_ref, wg_ref, wu_ref, wd_ref, out_ref):
    x = x_ref[...]
    g = jax.lax.dot_general(x, wg_ref[...], (((1,), (1,)), ((), ())),
                            preferred_element_type=jnp.float32)
    u = jax.lax.dot_general(x, wu_ref[...], (((1,), (1,)), ((), ())),
                            preferred_element_type=jnp.float32)
    h = g * jax.nn.sigmoid(g) * u
    out_ref[...] = jax.lax.dot_general(h, wd_ref[...], (((1,), (1,)), ((), ())),
                                       preferred_element_type=jnp.float32)


def _shared_mlp(x, sw_gate, sw_up, sw_down):
    bms = 256
    return pl.pallas_call(
        _shared_body,
        grid=(_T // bms,),
        in_specs=[
            pl.BlockSpec((bms, _H), lambda i: (i, 0)),
            pl.BlockSpec((_FS, _H), lambda i: (0, 0)),
            pl.BlockSpec((_FS, _H), lambda i: (0, 0)),
            pl.BlockSpec((_H, _FS), lambda i: (0, 0)),
        ],
        out_specs=pl.BlockSpec((bms, _H), lambda i: (i, 0)),
        out_shape=jax.ShapeDtypeStruct((_T, _H), jnp.float32),
    )(x, sw_gate, sw_up, sw_down)


def kernel(hidden_states, gate_w, w_gate, w_up, w_down, sw_gate, sw_up, sw_down):
    b, s, h = hidden_states.shape
    x = hidden_states.reshape(-1, h)

    pos, topk_w, teid = _gate_route(x, gate_w)

    xs = _dispatch(pos[:, 0], pos[:, 1], x)

    out_sorted = _moe_mlp(xs, w_gate, w_up, w_down, teid.reshape(-1)[:_NT])
    shared = _shared_mlp(x, sw_gate, sw_up, sw_down)

    y = (shared
         + topk_w[:, 0:1] * out_sorted[pos[:, 0]]
         + topk_w[:, 1:2] * out_sorted[pos[:, 1]])
    return y.reshape(b, s, h)


# speedup vs baseline: 1.2061x; 1.2061x over previous
import jax
import jax.numpy as jnp
from jax import lax
from jax.experimental import pallas as pl
from jax.experimental.pallas import tpu as pltpu
from jax.experimental.pallas import tpu_sc as plsc

_E = 8
_K = 2
_H = 1024
_F = 704
_FS = 1408
_T = 2048
_NP = _T * _K
_BM = 128
_NPAD = _NP + _E * _BM
_NT = _NPAD // _BM
_NTP = 48

_NC = 2
_NS = 16
_NW = _NC * _NS
_RW = _NPAD // _NW
_PFW = _NPAD // _NS
_SCW = _NP // _NS


def _gate_body(x_ref, gw_ref, pos_ref, wts_ref, teid_ref, s_ref):
    x = x_ref[...]
    gw = gw_ref[...]
    logits = jax.lax.dot_general(x, gw, (((1,), (1,)), ((), ())),
                                 preferred_element_type=jnp.float32)
    m = jnp.max(logits, axis=-1, keepdims=True)
    ex = jnp.exp(logits - m)
    scores = ex / jnp.sum(ex, axis=-1, keepdims=True)
    cols = jax.lax.broadcasted_iota(jnp.int32, scores.shape, 1)
    m1 = jnp.max(scores, axis=-1, keepdims=True)
    i1 = jnp.min(jnp.where(scores == m1, cols, _E), axis=-1, keepdims=True)
    masked = jnp.where(cols == i1, -jnp.inf, scores)
    m2 = jnp.max(masked, axis=-1, keepdims=True)
    i2 = jnp.min(jnp.where(masked == m2, cols, _E), axis=-1, keepdims=True)
    denom = m1 + m2 + 1e-20
    wts_ref[...] = jnp.concatenate([m1 / denom, m2 / denom], axis=1)

    oh0 = jnp.where(cols == i1, 1.0, 0.0)
    oh1 = jnp.where(cols == i2, 1.0, 0.0)
    both = oh0 + oh1

    cb = 128
    rr = jax.lax.broadcasted_iota(jnp.int32, (cb, cb), 0)
    cc = jax.lax.broadcasted_iota(jnp.int32, (cb, cb), 1)
    tri = jnp.where(rr > cc, 1.0, 0.0)
    off = jnp.zeros((1, _E), jnp.float32)
    for b in range(_T // cb):
        blk = both[b * cb:(b + 1) * cb, :]
        s_ref[pl.ds(b * cb, cb), :] = jax.lax.dot_general(
            tri, blk, (((1,), (0,)), ((), ())),
            preferred_element_type=jnp.float32) + off
        off = off + jnp.sum(blk, axis=0, keepdims=True)
    s = s_ref[...]

    counts = off
    padded = jnp.floor((counts + (_BM - 1)) / _BM) * _BM
    er = jax.lax.broadcasted_iota(jnp.int32, (_E, _E), 0)
    ec = jax.lax.broadcasted_iota(jnp.int32, (_E, _E), 1)
    triu = jnp.where(er <= ec, 1.0, 0.0)
    ends = jax.lax.dot_general(padded, triu, (((1,), (0,)), ((), ())),
                               preferred_element_type=jnp.float32)
    starts = ends - padded

    pos0 = jnp.sum(oh0 * (starts + s), axis=1, keepdims=True)
    pos1 = jnp.sum(oh1 * (starts + s + oh0), axis=1, keepdims=True)
    pos_ref[...] = jnp.concatenate([pos0, pos1], axis=1).astype(jnp.int32)

    tstart = (jax.lax.broadcasted_iota(jnp.int32, (_NTP, _E), 0)
              * _BM).astype(jnp.float32)
    acc = jnp.sum(jnp.where(tstart >= ends, 1.0, 0.0), axis=1)
    teid_ref[...] = jnp.minimum(acc, _E - 1).astype(jnp.int32).reshape(1, _NTP)


def _gate_route(x, gate_w):
    return pl.pallas_call(
        _gate_body,
        out_shape=(jax.ShapeDtypeStruct((_T, _K), jnp.int32),
                   jax.ShapeDtypeStruct((_T, _K), jnp.float32),
                   jax.ShapeDtypeStruct((1, _NTP), jnp.int32)),
        scratch_shapes=[pltpu.VMEM((_T, _E), jnp.float32)],
    )(x, gate_w)


_TW = _T // _NW


def _dispatch_body(p0_hbm, p1_hbm, x_hbm, xs_out, idx0_v, idx1_v, rows_v,
                   sem0, sem1):
    wid = lax.axis_index("c") * _NS + lax.axis_index("s")
    tb = wid * _TW
    pltpu.sync_copy(p0_hbm.at[pl.ds(tb, _TW)], idx0_v)
    pltpu.sync_copy(p1_hbm.at[pl.ds(tb, _TW)], idx1_v)
    pltpu.sync_copy(x_hbm.at[pl.ds(tb, _TW)], rows_v)
    d0 = pltpu.async_copy(rows_v, xs_out.at[idx0_v], sem0)
    d1 = pltpu.async_copy(rows_v, xs_out.at[idx1_v], sem1)
    d0.wait()
    d1.wait()


def _dispatch(pos0, pos1, x):
    mesh = plsc.VectorSubcoreMesh(core_axis_name="c", subcore_axis_name="s",
                                  num_cores=_NC, num_subcores=_NS)
    return pl.kernel(
        _dispatch_body,
        out_type=jax.ShapeDtypeStruct((_NPAD, _H), jnp.float32),
        mesh=mesh,
        scratch_types=[
            pltpu.VMEM((_TW,), jnp.int32),
            pltpu.VMEM((_TW,), jnp.int32),
            pltpu.VMEM((_TW, _H), jnp.float32),
            pltpu.SemaphoreType.DMA,
            pltpu.SemaphoreType.DMA,
        ],
        name="moe_dispatch_sc",
    )(pos0, pos1, x)


def _moe_body(tile_eid_ref, xs_ref, wg_ref, wu_ref, wd_ref, out_ref):
    x = xs_ref[...]
    g = jax.lax.dot_general(x, wg_ref[0], (((1,), (1,)), ((), ())),
                            preferred_element_type=jnp.float32)
    u = jax.lax.dot_general(x, wu_ref[0], (((1,), (1,)), ((), ())),
                            preferred_element_type=jnp.float32)
    h = g * jax.nn.sigmoid(g) * u
    out_ref[...] = jax.lax.dot_general(h, wd_ref[0], (((1,), (1,)), ((), ())),
                                       preferred_element_type=jnp.float32)


def _moe_mlp(xs, w_gate, w_up, w_down, tile_eid):
    grid_spec = pltpu.PrefetchScalarGridSpec(
        num_scalar_prefetch=1,
        grid=(_NT,),
        in_specs=[
            pl.BlockSpec((_BM, _H), lambda i, eid: (i, 0)),
            pl.BlockSpec((1, _F, _H), lambda i, eid: (eid[i], 0, 0)),
            pl.BlockSpec((1, _F, _H), lambda i, eid: (eid[i], 0, 0)),
            pl.BlockSpec((1, _H, _F), lambda i, eid: (eid[i], 0, 0)),
        ],
        out_specs=pl.BlockSpec((_BM, _H), lambda i, eid: (i, 0)),
    )
    return pl.pallas_call(
        _moe_body,
        grid_spec=grid_spec,
        out_shape=jax.ShapeDtypeStruct((_NPAD, _H), jnp.float32),
    )(tile_eid, xs, w_gate, w_up, w_down)


def _shared_body(x_ref, wg_ref, wu_ref, wd_ref, out_ref):
    x = x_ref[...]
    g = jax.lax.dot_general(x, wg_ref[...], (((1,), (1,)), ((), ())),
                            preferred_element_type=jnp.float32)
    u = jax.lax.dot_general(x, wu_ref[...], (((1,), (1,)), ((), ())),
                            preferred_element_type=jnp.float32)
    h = g * jax.nn.sigmoid(g) * u
    out_ref[...] = jax.lax.dot_general(h, wd_ref[...], (((1,), (1,)), ((), ())),
                                       preferred_element_type=jnp.float32)


def _shared_mlp(x, sw_gate, sw_up, sw_down):
    bms = 256
    return pl.pallas_call(
        _shared_body,
        grid=(_T // bms,),
        in_specs=[
            pl.BlockSpec((bms, _H), lambda i: (i, 0)),
            pl.BlockSpec((_FS, _H), lambda i: (0, 0)),
            pl.BlockSpec((_FS, _H), lambda i: (0, 0)),
            pl.BlockSpec((_H, _FS), lambda i: (0, 0)),
        ],
        out_specs=pl.BlockSpec((bms, _H), lambda i: (i, 0)),
        out_shape=jax.ShapeDtypeStruct((_T, _H), jnp.float32),
    )(x, sw_gate, sw_up, sw_down)


def kernel(hidden_states, gate_w, w_gate, w_up, w_down, sw_gate, sw_up, sw_down):
    b, s, h = hidden_states.shape
    x = hidden_states.reshape(-1, h)

    pos, topk_w, teid = _gate_route(x, gate_w)

    xs = _dispatch(pos[:, 0], pos[:, 1], x)

    out_sorted = _moe_mlp(xs, w_gate, w_up, w_down, teid.reshape(-1)[:_NT])
    shared = _shared_mlp(x, sw_gate, sw_up, sw_down)

    y = (shared
         + topk_w[:, 0:1] * out_sorted[pos[:, 0]]
         + topk_w[:, 1:2] * out_sorted[pos[:, 1]])
    return y.reshape(b, s, h)


# speedup vs baseline: 1.2064x; 1.0003x over previous
import jax
import jax.numpy as jnp
from jax import lax
from jax.experimental import pallas as pl
from jax.experimental.pallas import tpu as pltpu
from jax.experimental.pallas import tpu_sc as plsc

_E = 8
_K = 2
_H = 1024
_F = 704
_FS = 1408
_T = 2048
_NP = _T * _K
_BM = 128
_NPAD = _NP + _E * _BM
_NT = _NPAD // _BM
_NTP = 48

_NC = 2
_NS = 16
_NW = _NC * _NS
_RW = _NPAD // _NW
_PFW = _NPAD // _NS
_SCW = _NP // _NS


def _gate_body(x_ref, gw_ref, pos_ref, wts_ref, teid_ref, s_ref):
    x = x_ref[...]
    gw = gw_ref[...]
    logits = jax.lax.dot_general(x, gw, (((1,), (1,)), ((), ())),
                                 preferred_element_type=jnp.float32)
    m = jnp.max(logits, axis=-1, keepdims=True)
    ex = jnp.exp(logits - m)
    scores = ex / jnp.sum(ex, axis=-1, keepdims=True)
    cols = jax.lax.broadcasted_iota(jnp.int32, scores.shape, 1)
    m1 = jnp.max(scores, axis=-1, keepdims=True)
    i1 = jnp.min(jnp.where(scores == m1, cols, _E), axis=-1, keepdims=True)
    masked = jnp.where(cols == i1, -jnp.inf, scores)
    m2 = jnp.max(masked, axis=-1, keepdims=True)
    i2 = jnp.min(jnp.where(masked == m2, cols, _E), axis=-1, keepdims=True)
    denom = m1 + m2 + 1e-20
    wts_ref[...] = jnp.concatenate([m1 / denom, m2 / denom], axis=1)

    oh0 = jnp.where(cols == i1, 1.0, 0.0)
    oh1 = jnp.where(cols == i2, 1.0, 0.0)
    both = oh0 + oh1

    cb = 256
    rr = jax.lax.broadcasted_iota(jnp.int32, (cb, cb), 0)
    cc = jax.lax.broadcasted_iota(jnp.int32, (cb, cb), 1)
    tri = jnp.where(rr > cc, 1.0, 0.0)
    off = jnp.zeros((1, _E), jnp.float32)
    for b in range(_T // cb):
        blk = both[b * cb:(b + 1) * cb, :]
        s_ref[pl.ds(b * cb, cb), :] = jax.lax.dot_general(
            tri, blk, (((1,), (0,)), ((), ())),
            preferred_element_type=jnp.float32) + off
        off = off + jnp.sum(blk, axis=0, keepdims=True)
    s = s_ref[...]

    counts = off
    padded = jnp.floor((counts + (_BM - 1)) / _BM) * _BM
    er = jax.lax.broadcasted_iota(jnp.int32, (_E, _E), 0)
    ec = jax.lax.broadcasted_iota(jnp.int32, (_E, _E), 1)
    triu = jnp.where(er <= ec, 1.0, 0.0)
    ends = jax.lax.dot_general(padded, triu, (((1,), (0,)), ((), ())),
                               preferred_element_type=jnp.float32)
    starts = ends - padded

    pos0 = jnp.sum(oh0 * (starts + s), axis=1, keepdims=True)
    pos1 = jnp.sum(oh1 * (starts + s + oh0), axis=1, keepdims=True)
    pos_ref[...] = jnp.concatenate([pos0, pos1], axis=1).astype(jnp.int32)

    tstart = (jax.lax.broadcasted_iota(jnp.int32, (_NTP, _E), 0)
              * _BM).astype(jnp.float32)
    acc = jnp.sum(jnp.where(tstart >= ends, 1.0, 0.0), axis=1)
    teid_ref[...] = jnp.minimum(acc, _E - 1).astype(jnp.int32).reshape(1, _NTP)


def _gate_route(x, gate_w):
    return pl.pallas_call(
        _gate_body,
        out_shape=(jax.ShapeDtypeStruct((_T, _K), jnp.int32),
                   jax.ShapeDtypeStruct((_T, _K), jnp.float32),
                   jax.ShapeDtypeStruct((1, _NTP), jnp.int32)),
        scratch_shapes=[pltpu.VMEM((_T, _E), jnp.float32)],
    )(x, gate_w)


_TW = _T // _NW


def _dispatch_body(p0_hbm, p1_hbm, x_hbm, xs_out, idx0_v, idx1_v, rows_v,
                   sem0, sem1):
    wid = lax.axis_index("c") * _NS + lax.axis_index("s")
    tb = wid * _TW
    pltpu.sync_copy(p0_hbm.at[pl.ds(tb, _TW)], idx0_v)
    pltpu.sync_copy(p1_hbm.at[pl.ds(tb, _TW)], idx1_v)
    pltpu.sync_copy(x_hbm.at[pl.ds(tb, _TW)], rows_v)
    d0 = pltpu.async_copy(rows_v, xs_out.at[idx0_v], sem0)
    d1 = pltpu.async_copy(rows_v, xs_out.at[idx1_v], sem1)
    d0.wait()
    d1.wait()


def _dispatch(pos0, pos1, x):
    mesh = plsc.VectorSubcoreMesh(core_axis_name="c", subcore_axis_name="s",
                                  num_cores=_NC, num_subcores=_NS)
    return pl.kernel(
        _dispatch_body,
        out_type=jax.ShapeDtypeStruct((_NPAD, _H), jnp.float32),
        mesh=mesh,
        scratch_types=[
            pltpu.VMEM((_TW,), jnp.int32),
            pltpu.VMEM((_TW,), jnp.int32),
            pltpu.VMEM((_TW, _H), jnp.float32),
            pltpu.SemaphoreType.DMA,
            pltpu.SemaphoreType.DMA,
        ],
        name="moe_dispatch_sc",
    )(pos0, pos1, x)


def _moe_body(tile_eid_ref, xs_ref, wg_ref, wu_ref, wd_ref, out_ref):
    x = xs_ref[...]
    g = jax.lax.dot_general(x, wg_ref[0], (((1,), (1,)), ((), ())),
                            preferred_element_type=jnp.float32)
    u = jax.lax.dot_general(x, wu_ref[0], (((1,), (1,)), ((), ())),
                            preferred_element_type=jnp.float32)
    h = g * jax.nn.sigmoid(g) * u
    out_ref[...] = jax.lax.dot_general(h, wd_ref[0], (((1,), (1,)), ((), ())),
                                       preferred_element_type=jnp.float32)


def _moe_mlp(xs, w_gate, w_up, w_down, tile_eid):
    grid_spec = pltpu.PrefetchScalarGridSpec(
        num_scalar_prefetch=1,
        grid=(_NT,),
        in_specs=[
            pl.BlockSpec((_BM, _H), lambda i, eid: (i, 0)),
            pl.BlockSpec((1, _F, _H), lambda i, eid: (eid[i], 0, 0)),
            pl.BlockSpec((1, _F, _H), lambda i, eid: (eid[i], 0, 0)),
            pl.BlockSpec((1, _H, _F), lambda i, eid: (eid[i], 0, 0)),
        ],
        out_specs=pl.BlockSpec((_BM, _H), lambda i, eid: (i, 0)),
    )
    return pl.pallas_call(
        _moe_body,
        grid_spec=grid_spec,
        out_shape=jax.ShapeDtypeStruct((_NPAD, _H), jnp.float32),
    )(tile_eid, xs, w_gate, w_up, w_down)


def _shared_body(x_ref, wg_ref, wu_ref, wd_ref, out_ref):
    x = x_ref[...]
    g = jax.lax.dot_general(x, wg_ref[...], (((1,), (1,)), ((), ())),
                            preferred_element_type=jnp.float32)
    u = jax.lax.dot_general(x, wu_ref[...], (((1,), (1,)), ((), ())),
                            preferred_element_type=jnp.float32)
    h = g * jax.nn.sigmoid(g) * u
    out_ref[...] = jax.lax.dot_general(h, wd_ref[...], (((1,), (1,)), ((), ())),
                                       preferred_element_type=jnp.float32)


def _shared_mlp(x, sw_gate, sw_up, sw_down):
    bms = 256
    return pl.pallas_call(
        _shared_body,
        grid=(_T // bms,),
        in_specs=[
            pl.BlockSpec((bms, _H), lambda i: (i, 0)),
            pl.BlockSpec((_FS, _H), lambda i: (0, 0)),
            pl.BlockSpec((_FS, _H), lambda i: (0, 0)),
            pl.BlockSpec((_H, _FS), lambda i: (0, 0)),
        ],
        out_specs=pl.BlockSpec((bms, _H), lambda i: (i, 0)),
        out_shape=jax.ShapeDtypeStruct((_T, _H), jnp.float32),
    )(x, sw_gate, sw_up, sw_down)


def kernel(hidden_states, gate_w, w_gate, w_up, w_down, sw_gate, sw_up, sw_down):
    b, s, h = hidden_states.shape
    x = hidden_states.reshape(-1, h)

    pos, topk_w, teid = _gate_route(x, gate_w)

    xs = _dispatch(pos[:, 0], pos[:, 1], x)

    out_sorted = _moe_mlp(xs, w_gate, w_up, w_down, teid.reshape(-1)[:_NT])
    shared = _shared_mlp(x, sw_gate, sw_up, sw_down)

    y = (shared
         + topk_w[:, 0:1] * out_sorted[pos[:, 0]]
         + topk_w[:, 1:2] * out_sorted[pos[:, 1]])
    return y.reshape(b, s, h)
